# baseline (speedup 1.0000x reference)
import functools
import jax
import jax.numpy as jnp
from jax import lax
from jax.experimental import pallas as pl
from jax.experimental.pallas import tpu as pltpu
from jax.experimental.pallas import tpu_sc as plsc

B, N, M, CC, CP, COUT = 8, 8192, 1024, 256, 256, 256
CIN = CC + CP
BN = 512
NB = N // BN
BN2 = 2048
NB2 = N // BN2

NW = 32
PPW = (B * N) // NW
G = 64
NG = PPW // G
CCH = CC // 16


def _knn_kernel(pc_ref, ccT_ref, idx_ref, w_ref):
    b_i = pl.program_id(0)

    c3 = ccT_ref[0]
    p = pc_ref[0]
    p2 = jnp.zeros((1, BN), jnp.float32)
    c2 = jnp.zeros((M, 1), jnp.float32)
    cross = jnp.zeros((M, BN), jnp.float32)
    for d in range(3):
        cd = c3[:, d:d + 1]
        pd = p[d:d + 1, :]
        p2 = p2 + pd * pd
        c2 = c2 + cd * cd
        cdb = cd.astype(jnp.bfloat16).astype(jnp.float32)
        pdb = pd.astype(jnp.bfloat16).astype(jnp.float32)
        cross = cross + cdb * pdb
    d2 = jnp.maximum((p2 + c2) - 2.0 * cross, 0.0)

    iota_m = jax.lax.broadcasted_iota(jnp.int32, (M, BN), 0)
    inf = jnp.float32(3.0e38)
    cur = d2
    ids = []
    wks = []
    for k in range(3):
        v = jnp.min(cur, axis=0, keepdims=True)
        i = jnp.min(jnp.where(cur == v, iota_m, M), axis=0,
                    keepdims=True)
        ids.append(i)
        wks.append(1.0 / (jnp.sqrt(jnp.maximum(v, 1e-12)) + 1e-8))
        if k < 2:
            cur = jnp.where(iota_m == i, inf, cur)
    wsum = wks[0] + wks[1] + wks[2]
    for k in range(3):
        idx_ref[0, k:k + 1, :] = ids[k] + b_i * M
        w_ref[0, k:k + 1, :] = wks[k] / wsum


@functools.partial(
    pl.kernel,
    mesh=plsc.VectorSubcoreMesh(core_axis_name="c", subcore_axis_name="s"),
    out_type=jax.ShapeDtypeStruct((B * N, CC), jnp.float32),
    scratch_types=[
        pltpu.VMEM((3, PPW), jnp.int32),
        pltpu.VMEM((3, PPW), jnp.float32),
        pltpu.VMEM((3, G, CC), jnp.float32),
        pltpu.VMEM((G, CC), jnp.float32),
        pltpu.SemaphoreType.DMA,
    ],
)
def _sc_gather(idx_hbm, w_hbm, cfT_hbm, out_hbm, idxv, wv, rows, outv, sem):
    wid = lax.axis_index("s") * 2 + lax.axis_index("c")
    pbase = wid * PPW
    bi = pbase // N
    off = pbase - bi * N
    pltpu.sync_copy(idx_hbm.at[bi, :, pl.ds(off, PPW)], idxv)
    pltpu.sync_copy(w_hbm.at[bi, :, pl.ds(off, PPW)], wv)

    def chunk(g, carry):
        jb = g * G
        for k in range(3):
            pltpu.async_copy(
                cfT_hbm.at[idxv.at[k, pl.ds(jb, G)]], rows.at[k], sem
            ).wait()

        def group(g2, c2):
            wv0 = wv[0, pl.ds(jb + g2 * 16, 16)]
            wv1 = wv[1, pl.ds(jb + g2 * 16, 16)]
            wv2 = wv[2, pl.ds(jb + g2 * 16, 16)]
            for jj in range(16):
                w0 = wv0[jj]
                w1 = wv1[jj]
                w2 = wv2[jj]
                pt = g2 * 16 + jj
                for c in range(CCH):
                    sl = pl.ds(c * 16, 16)
                    outv[pt, sl] = (rows[0, pt, sl] * w0
                                    + rows[1, pt, sl] * w1
                                    + rows[2, pt, sl] * w2)
            return c2

        lax.fori_loop(0, G // 16, group, 0)
        pltpu.sync_copy(outv, out_hbm.at[pl.ds(pbase + jb, G)])
        return carry

    lax.fori_loop(0, NG, chunk, 0)


def _mlp_kernel(it_ref, pf_ref, w_ref, b_ref, y_ref, ps_ref, pss_ref):
    it = it_ref[...]
    y = lax.dot_general(w_ref[:, :CC], it,
                        (((1,), (1,)), ((), ())),
                        preferred_element_type=jnp.float32)
    y = y + jnp.dot(w_ref[:, CC:], pf_ref[0],
                    preferred_element_type=jnp.float32)
    y = y + b_ref[...]
    y_ref[0] = y
    ps_ref[0] = jnp.sum(y, axis=1, keepdims=True)
    pss_ref[0] = jnp.sum(y * y, axis=1, keepdims=True)


def _bn_kernel(y_ref, a_ref, c_ref, o_ref):
    o_ref[0] = jnp.maximum(y_ref[0] * a_ref[...] + c_ref[...], 0.0)


def kernel(points_coords, centers_coords, centers_features, points_features,
           W, b, gamma, beta):
    ccT = centers_coords.transpose(0, 2, 1)
    cfT = centers_features.transpose(0, 2, 1).reshape(B * M, CC)
    b2 = b.reshape(COUT, 1)

    idx, wts = pl.pallas_call(
        _knn_kernel,
        grid=(B, NB),
        in_specs=[
            pl.BlockSpec((1, 3, BN), lambda bi, nb: (bi, 0, nb)),
            pl.BlockSpec((1, M, 3), lambda bi, nb: (bi, 0, 0)),
        ],
        out_specs=[
            pl.BlockSpec((1, 3, BN), lambda bi, nb: (bi, 0, nb)),
            pl.BlockSpec((1, 3, BN), lambda bi, nb: (bi, 0, nb)),
        ],
        out_shape=[
            jax.ShapeDtypeStruct((B, 3, N), jnp.int32),
            jax.ShapeDtypeStruct((B, 3, N), jnp.float32),
        ],
    )(points_coords, ccT)

    interp = _sc_gather(idx, wts, cfT)

    y, ps, pss = pl.pallas_call(
        _mlp_kernel,
        grid=(B, NB),
        in_specs=[
            pl.BlockSpec((BN, CC), lambda bi, nb: (bi * NB + nb, 0)),
            pl.BlockSpec((1, CP, BN), lambda bi, nb: (bi, 0, nb)),
            pl.BlockSpec((COUT, CIN), lambda bi, nb: (0, 0)),
            pl.BlockSpec((COUT, 1), lambda bi, nb: (0, 0)),
        ],
        out_specs=[
            pl.BlockSpec((1, COUT, BN), lambda bi, nb: (bi, 0, nb)),
            pl.BlockSpec((1, COUT, 1), lambda bi, nb: (bi * NB + nb, 0, 0)),
            pl.BlockSpec((1, COUT, 1), lambda bi, nb: (bi * NB + nb, 0, 0)),
        ],
        out_shape=[
            jax.ShapeDtypeStruct((B, COUT, N), jnp.float32),
            jax.ShapeDtypeStruct((B * NB, COUT, 1), jnp.float32),
            jax.ShapeDtypeStruct((B * NB, COUT, 1), jnp.float32),
        ],
    )(interp, points_features, W, b2)

    cnt = jnp.float32(B * N)
    s = jnp.sum(ps[:, :, 0], axis=0)
    ss = jnp.sum(pss[:, :, 0], axis=0)
    mean = s / cnt
    var = ss / cnt - mean * mean
    a = gamma / jnp.sqrt(var + 1e-5)
    cshift = beta - mean * a

    out = pl.pallas_call(
        _bn_kernel,
        grid=(B, NB2),
        in_specs=[
            pl.BlockSpec((1, COUT, BN2), lambda bi, nb: (bi, 0, nb)),
            pl.BlockSpec((COUT, 1), lambda bi, nb: (0, 0)),
            pl.BlockSpec((COUT, 1), lambda bi, nb: (0, 0)),
        ],
        out_specs=pl.BlockSpec((1, COUT, BN2), lambda bi, nb: (bi, 0, nb)),
        out_shape=jax.ShapeDtypeStruct((B, COUT, N), jnp.float32),
    )(y, a.reshape(COUT, 1), cshift.reshape(COUT, 1))

    return (out, points_coords)


# speedup vs baseline: 1.0314x; 1.0314x over previous
import functools
import jax
import jax.numpy as jnp
from jax import lax
from jax.experimental import pallas as pl
from jax.experimental.pallas import tpu as pltpu
from jax.experimental.pallas import tpu_sc as plsc

B, N, M, CC, CP, COUT = 8, 8192, 1024, 256, 256, 256
CIN = CC + CP
KBN = 1024
KNB = N // KBN
BN = 512
NB = N // BN

NW = 32
PPW = (B * N) // NW
G = 32
NG = PPW // G
CCH = CC // 16


def _knn_kernel(pc_ref, ccT_ref, idx_ref, w_ref):
    b_i = pl.program_id(0)

    c3 = ccT_ref[0]
    p = pc_ref[0]
    p2 = jnp.zeros((1, KBN), jnp.float32)
    c2 = jnp.zeros((M, 1), jnp.float32)
    cross = jnp.zeros((M, KBN), jnp.float32)
    for d in range(3):
        cd = c3[:, d:d + 1]
        pd = p[d:d + 1, :]
        p2 = p2 + pd * pd
        c2 = c2 + cd * cd
        cdb = cd.astype(jnp.bfloat16).astype(jnp.float32)
        pdb = pd.astype(jnp.bfloat16).astype(jnp.float32)
        cross = cross + cdb * pdb
    d2 = jnp.maximum((p2 + c2) - 2.0 * cross, 0.0)

    iota_m = jax.lax.broadcasted_iota(jnp.int32, (M, KBN), 0)
    inf = jnp.float32(3.0e38)
    cur = d2
    ids = []
    wks = []
    for k in range(3):
        v = jnp.min(cur, axis=0, keepdims=True)
        i = jnp.min(jnp.where(cur == v, iota_m, M), axis=0,
                    keepdims=True)
        ids.append(i)
        wks.append(1.0 / (jnp.sqrt(jnp.maximum(v, 1e-12)) + 1e-8))
        if k < 2:
            cur = jnp.where(iota_m == i, inf, cur)
    wsum = wks[0] + wks[1] + wks[2]
    for k in range(3):
        idx_ref[0, k:k + 1, :] = ids[k] + b_i * M
        w_ref[0, k:k + 1, :] = wks[k] / wsum


@functools.partial(
    pl.kernel,
    mesh=plsc.VectorSubcoreMesh(core_axis_name="c", subcore_axis_name="s"),
    out_type=jax.ShapeDtypeStruct((B * N, CC), jnp.float32),
    scratch_types=[
        pltpu.VMEM((3, PPW), jnp.int32),
        pltpu.VMEM((3, PPW), jnp.float32),
        pltpu.VMEM((2, 3, G, CC), jnp.float32),
        pltpu.VMEM((G, CC), jnp.float32),
        pltpu.SemaphoreType.DMA,
        pltpu.SemaphoreType.DMA,
    ],
)
def _sc_gather(idx_hbm, w_hbm, cfT_hbm, out_hbm, idxv, wv, rows, outv,
               sem0, sem1):
    wid = lax.axis_index("s") * 2 + lax.axis_index("c")
    pbase = wid * PPW
    bi = pbase // N
    off = pbase - bi * N
    pltpu.sync_copy(idx_hbm.at[bi, :, pl.ds(off, PPW)], idxv)
    pltpu.sync_copy(w_hbm.at[bi, :, pl.ds(off, PPW)], wv)
    sems = (sem0, sem1)

    def fire(g, buf):
        jb = g * G
        for k in range(3):
            pltpu.async_copy(cfT_hbm.at[idxv.at[k, pl.ds(jb, G)]],
                             rows.at[buf, k], sems[buf])

    def drain(buf):
        for k in range(3):
            pltpu.make_async_copy(cfT_hbm.at[pl.ds(0, G)],
                                  rows.at[buf, k], sems[buf]).wait()

    def compute(g, buf):
        jb = g * G

        def group(g2, c2):
            base = jb + g2 * 16
            wv0 = wv[0, pl.ds(base, 16)]
            wv1 = wv[1, pl.ds(base, 16)]
            wv2 = wv[2, pl.ds(base, 16)]
            for jj in range(16):
                pt = g2 * 16 + jj
                for c in range(CCH):
                    sl = pl.ds(c * 16, 16)
                    outv[pt, sl] = (rows[buf, 0, pt, sl] * wv0[jj]
                                    + rows[buf, 1, pt, sl] * wv1[jj]
                                    + rows[buf, 2, pt, sl] * wv2[jj])
            return c2

        lax.fori_loop(0, G // 16, group, 0)
        pltpu.sync_copy(outv, out_hbm.at[pl.ds(pbase + jb, G)])

    fire(0, 0)

    def pair(h, carry):
        g0 = h * 2
        fire(g0 + 1, 1)
        drain(0)
        compute(g0, 0)

        @pl.when(g0 + 2 < NG)
        def _():
            fire(g0 + 2, 0)

        drain(1)
        compute(g0 + 1, 1)
        return carry

    lax.fori_loop(0, NG // 2, pair, 0)


def _mlp_kernel(it_ref, pfT_ref, wt_ref, b_ref, y_ref, ps_ref, pss_ref):
    it = it_ref[...]
    pf = pfT_ref[0]
    y = jnp.dot(it, wt_ref[:CC, :], preferred_element_type=jnp.float32)
    y = y + jnp.dot(pf, wt_ref[CC:, :], preferred_element_type=jnp.float32)
    y = y + b_ref[...]
    y_ref[0] = y
    ps_ref[0] = jnp.sum(y, axis=0, keepdims=True)
    pss_ref[0] = jnp.sum(y * y, axis=0, keepdims=True)


def _bn_kernel(y_ref, ac_ref, o_ref):
    yv = y_ref[0]
    z = jnp.maximum(yv * ac_ref[0:1, :] + ac_ref[1:2, :], 0.0)
    o_ref[0] = z.T


def kernel(points_coords, centers_coords, centers_features, points_features,
           W, b, gamma, beta):
    ccT = centers_coords.transpose(0, 2, 1)
    cfT = centers_features.transpose(0, 2, 1).reshape(B * M, CC)
    pfT = points_features.transpose(0, 2, 1)
    WT = W.T
    b_row = b.reshape(1, COUT)

    idx, wts = pl.pallas_call(
        _knn_kernel,
        grid=(B, KNB),
        in_specs=[
            pl.BlockSpec((1, 3, KBN), lambda bi, nb: (bi, 0, nb)),
            pl.BlockSpec((1, M, 3), lambda bi, nb: (bi, 0, 0)),
        ],
        out_specs=[
            pl.BlockSpec((1, 3, KBN), lambda bi, nb: (bi, 0, nb)),
            pl.BlockSpec((1, 3, KBN), lambda bi, nb: (bi, 0, nb)),
        ],
        out_shape=[
            jax.ShapeDtypeStruct((B, 3, N), jnp.int32),
            jax.ShapeDtypeStruct((B, 3, N), jnp.float32),
        ],
    )(points_coords, ccT)

    interp = _sc_gather(idx, wts, cfT)

    yT, ps, pss = pl.pallas_call(
        _mlp_kernel,
        grid=(B, NB),
        in_specs=[
            pl.BlockSpec((BN, CC), lambda bi, nb: (bi * NB + nb, 0)),
            pl.BlockSpec((1, BN, CP), lambda bi, nb: (bi, nb, 0)),
            pl.BlockSpec((CIN, COUT), lambda bi, nb: (0, 0)),
            pl.BlockSpec((1, COUT), lambda bi, nb: (0, 0)),
        ],
        out_specs=[
            pl.BlockSpec((1, BN, COUT), lambda bi, nb: (bi, nb, 0)),
            pl.BlockSpec((1, 1, COUT), lambda bi, nb: (bi * NB + nb, 0, 0)),
            pl.BlockSpec((1, 1, COUT), lambda bi, nb: (bi * NB + nb, 0, 0)),
        ],
        out_shape=[
            jax.ShapeDtypeStruct((B, N, COUT), jnp.float32),
            jax.ShapeDtypeStruct((B * NB, 1, COUT), jnp.float32),
            jax.ShapeDtypeStruct((B * NB, 1, COUT), jnp.float32),
        ],
    )(interp, pfT, WT, b_row)

    cnt = jnp.float32(B * N)
    s = jnp.sum(ps[:, 0, :], axis=0)
    ss = jnp.sum(pss[:, 0, :], axis=0)
    mean = s / cnt
    var = ss / cnt - mean * mean
    a = gamma / jnp.sqrt(var + 1e-5)
    cshift = beta - mean * a
    ac = jnp.concatenate([a.reshape(1, COUT), cshift.reshape(1, COUT)], 0)

    out = pl.pallas_call(
        _bn_kernel,
        grid=(B, NB),
        in_specs=[
            pl.BlockSpec((1, BN, COUT), lambda bi, nb: (bi, nb, 0)),
            pl.BlockSpec((2, COUT), lambda bi, nb: (0, 0)),
        ],
        out_specs=pl.BlockSpec((1, COUT, BN), lambda bi, nb: (bi, 0, nb)),
        out_shape=jax.ShapeDtypeStruct((B, COUT, N), jnp.float32),
    )(yT, ac)

    return (out, points_coords)


# speedup vs baseline: 1.0424x; 1.0107x over previous
import functools
import jax
import jax.numpy as jnp
from jax import lax
from jax.experimental import pallas as pl
from jax.experimental.pallas import tpu as pltpu
from jax.experimental.pallas import tpu_sc as plsc

B, N, M, CC, CP, COUT = 8, 8192, 1024, 256, 256, 256
CIN = CC + CP
KBN = 1024
KNB = N // KBN
BN = 512
NB = N // BN

NW = 32
PPW = (B * N) // NW
G = 32
NG = PPW // G
CCH = CC // 16


def _knn_kernel(pc_ref, ccT_ref, idx_ref, w_ref):
    b_i = pl.program_id(0)

    c3 = ccT_ref[0]
    p = pc_ref[0]
    p2 = jnp.zeros((1, KBN), jnp.float32)
    c2 = jnp.zeros((M, 1), jnp.float32)
    cross = jnp.zeros((M, KBN), jnp.float32)
    for d in range(3):
        cd = c3[:, d:d + 1]
        pd = p[d:d + 1, :]
        p2 = p2 + pd * pd
        c2 = c2 + cd * cd
        cdb = cd.astype(jnp.bfloat16).astype(jnp.float32)
        pdb = pd.astype(jnp.bfloat16).astype(jnp.float32)
        cross = cross + cdb * pdb
    d2 = jnp.maximum((p2 + c2) - 2.0 * cross, 0.0)

    iota_m = jax.lax.broadcasted_iota(jnp.int32, (M, KBN), 0)
    inf = jnp.float32(3.0e38)
    cur = d2
    ids = []
    wks = []
    for k in range(3):
        v = jnp.min(cur, axis=0, keepdims=True)
        i = jnp.min(jnp.where(cur == v, iota_m, M), axis=0,
                    keepdims=True)
        ids.append(i)
        wks.append(1.0 / (jnp.sqrt(jnp.maximum(v, 1e-12)) + 1e-8))
        if k < 2:
            cur = jnp.where(iota_m == i, inf, cur)
    wsum = wks[0] + wks[1] + wks[2]
    for k in range(3):
        idx_ref[0, k:k + 1, :] = ids[k] + b_i * M
        w_ref[0, k:k + 1, :] = wks[k] / wsum


@functools.partial(
    pl.kernel,
    mesh=plsc.VectorSubcoreMesh(core_axis_name="c", subcore_axis_name="s"),
    out_type=jax.ShapeDtypeStruct((B * N, CC), jnp.float32),
    scratch_types=[
        pltpu.VMEM((3, PPW), jnp.int32),
        pltpu.VMEM((3, PPW), jnp.float32),
        pltpu.VMEM((2, 3, G, CC), jnp.float32),
        pltpu.VMEM((2, G, CC), jnp.float32),
        pltpu.SemaphoreType.DMA,
        pltpu.SemaphoreType.DMA,
        pltpu.SemaphoreType.DMA,
        pltpu.SemaphoreType.DMA,
    ],
)
def _sc_gather(idx_hbm, w_hbm, cfT_hbm, out_hbm, idxv, wv, rows, outv,
               sem0, sem1, osem0, osem1):
    wid = lax.axis_index("s") * 2 + lax.axis_index("c")
    pbase = wid * PPW
    bi = pbase // N
    off = pbase - bi * N
    pltpu.sync_copy(idx_hbm.at[bi, :, pl.ds(off, PPW)], idxv)
    pltpu.sync_copy(w_hbm.at[bi, :, pl.ds(off, PPW)], wv)
    sems = (sem0, sem1)
    osems = (osem0, osem1)

    def fire(g, buf):
        jb = g * G
        for k in range(3):
            pltpu.async_copy(cfT_hbm.at[idxv.at[k, pl.ds(jb, G)]],
                             rows.at[buf, k], sems[buf])

    def drain(buf):
        for k in range(3):
            pltpu.make_async_copy(cfT_hbm.at[pl.ds(0, G)],
                                  rows.at[buf, k], sems[buf]).wait()

    def owait(buf):
        pltpu.make_async_copy(outv.at[buf], out_hbm.at[pl.ds(0, G)],
                              osems[buf]).wait()

    def compute(g, buf):
        jb = g * G

        def group(g2, c2):
            base = jb + g2 * 16
            wv0 = wv[0, pl.ds(base, 16)]
            wv1 = wv[1, pl.ds(base, 16)]
            wv2 = wv[2, pl.ds(base, 16)]
            for jj in range(16):
                pt = g2 * 16 + jj
                for c in range(CCH):
                    sl = pl.ds(c * 16, 16)
                    outv[buf, pt, sl] = (rows[buf, 0, pt, sl] * wv0[jj]
                                         + rows[buf, 1, pt, sl] * wv1[jj]
                                         + rows[buf, 2, pt, sl] * wv2[jj])
            return c2

        lax.fori_loop(0, G // 16, group, 0)
        pltpu.async_copy(outv.at[buf], out_hbm.at[pl.ds(pbase + jb, G)],
                         osems[buf])

    fire(0, 0)

    def pair(h, carry):
        g0 = h * 2
        fire(g0 + 1, 1)
        drain(0)

        @pl.when(h > 0)
        def _():
            owait(0)

        compute(g0, 0)

        @pl.when(g0 + 2 < NG)
        def _():
            fire(g0 + 2, 0)

        drain(1)

        @pl.when(h > 0)
        def _():
            owait(1)

        compute(g0 + 1, 1)
        return carry

    lax.fori_loop(0, NG // 2, pair, 0)
    owait(0)
    owait(1)


def _mlp_kernel(it_ref, pfT_ref, wt_ref, b_ref, y_ref, ps_ref, pss_ref):
    it = it_ref[...]
    pf = pfT_ref[0]
    y = jnp.dot(it, wt_ref[:CC, :], preferred_element_type=jnp.float32)
    y = y + jnp.dot(pf, wt_ref[CC:, :], preferred_element_type=jnp.float32)
    y = y + b_ref[...]
    y_ref[0] = y
    ps_ref[0] = jnp.sum(y, axis=0, keepdims=True)
    pss_ref[0] = jnp.sum(y * y, axis=0, keepdims=True)


def _bn_kernel(y_ref, ac_ref, o_ref):
    yv = y_ref[0]
    z = jnp.maximum(yv * ac_ref[0:1, :] + ac_ref[1:2, :], 0.0)
    o_ref[0] = z.T


def kernel(points_coords, centers_coords, centers_features, points_features,
           W, b, gamma, beta):
    ccT = centers_coords.transpose(0, 2, 1)
    cfT = centers_features.transpose(0, 2, 1).reshape(B * M, CC)
    pfT = points_features.transpose(0, 2, 1)
    WT = W.T
    b_row = b.reshape(1, COUT)

    idx, wts = pl.pallas_call(
        _knn_kernel,
        grid=(B, KNB),
        in_specs=[
            pl.BlockSpec((1, 3, KBN), lambda bi, nb: (bi, 0, nb)),
            pl.BlockSpec((1, M, 3), lambda bi, nb: (bi, 0, 0)),
        ],
        out_specs=[
            pl.BlockSpec((1, 3, KBN), lambda bi, nb: (bi, 0, nb)),
            pl.BlockSpec((1, 3, KBN), lambda bi, nb: (bi, 0, nb)),
        ],
        out_shape=[
            jax.ShapeDtypeStruct((B, 3, N), jnp.int32),
            jax.ShapeDtypeStruct((B, 3, N), jnp.float32),
        ],
    )(points_coords, ccT)

    interp = _sc_gather(idx, wts, cfT)

    yT, ps, pss = pl.pallas_call(
        _mlp_kernel,
        grid=(B, NB),
        in_specs=[
            pl.BlockSpec((BN, CC), lambda bi, nb: (bi * NB + nb, 0)),
            pl.BlockSpec((1, BN, CP), lambda bi, nb: (bi, nb, 0)),
            pl.BlockSpec((CIN, COUT), lambda bi, nb: (0, 0)),
            pl.BlockSpec((1, COUT), lambda bi, nb: (0, 0)),
        ],
        out_specs=[
            pl.BlockSpec((1, BN, COUT), lambda bi, nb: (bi, nb, 0)),
            pl.BlockSpec((1, 1, COUT), lambda bi, nb: (bi * NB + nb, 0, 0)),
            pl.BlockSpec((1, 1, COUT), lambda bi, nb: (bi * NB + nb, 0, 0)),
        ],
        out_shape=[
            jax.ShapeDtypeStruct((B, N, COUT), jnp.float32),
            jax.ShapeDtypeStruct((B * NB, 1, COUT), jnp.float32),
            jax.ShapeDtypeStruct((B * NB, 1, COUT), jnp.float32),
        ],
    )(interp, pfT, WT, b_row)

    cnt = jnp.float32(B * N)
    s = jnp.sum(ps[:, 0, :], axis=0)
    ss = jnp.sum(pss[:, 0, :], axis=0)
    mean = s / cnt
    var = ss / cnt - mean * mean
    a = gamma / jnp.sqrt(var + 1e-5)
    cshift = beta - mean * a
    ac = jnp.concatenate([a.reshape(1, COUT), cshift.reshape(1, COUT)], 0)

    out = pl.pallas_call(
        _bn_kernel,
        grid=(B, NB),
        in_specs=[
            pl.BlockSpec((1, BN, COUT), lambda bi, nb: (bi, nb, 0)),
            pl.BlockSpec((2, COUT), lambda bi, nb: (0, 0)),
        ],
        out_specs=pl.BlockSpec((1, COUT, BN), lambda bi, nb: (bi, 0, nb)),
        out_shape=jax.ShapeDtypeStruct((B, COUT, N), jnp.float32),
    )(yT, ac)

    return (out, points_coords)


# speedup vs baseline: 1.2019x; 1.1530x over previous
import functools
import jax
import jax.numpy as jnp
from jax import lax
from jax.experimental import pallas as pl
from jax.experimental.pallas import tpu as pltpu
from jax.experimental.pallas import tpu_sc as plsc

B, N, M, CC, CP, COUT = 8, 8192, 1024, 256, 256, 256
CIN = CC + CP
KBN = 1024
KNB = N // KBN
BN = 512
NB = N // BN

NW = 32
PPW = (B * N) // NW
G = 64
NG = PPW // G


def _knn_kernel(pc_ref, ccT_ref, idx_ref, w_ref):
    b_i = pl.program_id(0)

    c3 = ccT_ref[0]
    p = pc_ref[0]
    p2 = jnp.zeros((1, KBN), jnp.float32)
    c2 = jnp.zeros((M, 1), jnp.float32)
    cross = jnp.zeros((M, KBN), jnp.float32)
    for d in range(3):
        cd = c3[:, d:d + 1]
        pd = p[d:d + 1, :]
        p2 = p2 + pd * pd
        c2 = c2 + cd * cd
        cdb = cd.astype(jnp.bfloat16).astype(jnp.float32)
        pdb = pd.astype(jnp.bfloat16).astype(jnp.float32)
        cross = cross + cdb * pdb
    d2 = jnp.maximum((p2 + c2) - 2.0 * cross, 0.0)

    iota_m = jax.lax.broadcasted_iota(jnp.int32, (M, KBN), 0)
    inf = jnp.float32(3.0e38)
    cur = d2
    ids = []
    wks = []
    for k in range(3):
        v = jnp.min(cur, axis=0, keepdims=True)
        i = jnp.min(jnp.where(cur == v, iota_m, M), axis=0,
                    keepdims=True)
        ids.append(i)
        wks.append(1.0 / (jnp.sqrt(jnp.maximum(v, 1e-12)) + 1e-8))
        if k < 2:
            cur = jnp.where(iota_m == i, inf, cur)
    wsum = wks[0] + wks[1] + wks[2]
    for k in range(3):
        idx_ref[0, k:k + 1, :] = ids[k] + b_i * M
        w_ref[0, k:k + 1, :] = wks[k] / wsum


def _z_kernel(cf_ref, wcT_ref, z_ref):
    z_ref[...] = jnp.dot(cf_ref[...], wcT_ref[...],
                         preferred_element_type=jnp.float32)


@functools.partial(
    pl.kernel,
    mesh=plsc.VectorSubcoreMesh(core_axis_name="c", subcore_axis_name="s"),
    out_type=jax.ShapeDtypeStruct((3, B * N, COUT), jnp.float32),
    scratch_types=[
        pltpu.VMEM((3, PPW), jnp.int32),
        pltpu.VMEM((2, 3, G, COUT), jnp.float32),
        pltpu.SemaphoreType.DMA,
        pltpu.SemaphoreType.DMA,
        pltpu.SemaphoreType.DMA,
        pltpu.SemaphoreType.DMA,
    ],
)
def _sc_gather(idx_hbm, z_hbm, out_hbm, idxv, rows, sem0, sem1, osem0, osem1):
    wid = lax.axis_index("s") * 2 + lax.axis_index("c")
    pbase = wid * PPW
    bi = pbase // N
    off = pbase - bi * N
    pltpu.sync_copy(idx_hbm.at[bi, :, pl.ds(off, PPW)], idxv)
    sems = (sem0, sem1)
    osems = (osem0, osem1)

    def fire_in(g, buf):
        jb = g * G
        for k in range(3):
            pltpu.async_copy(z_hbm.at[idxv.at[k, pl.ds(jb, G)]],
                             rows.at[buf, k], sems[buf])

    def drain_in(buf):
        for k in range(3):
            pltpu.make_async_copy(z_hbm.at[pl.ds(0, G)],
                                  rows.at[buf, k], sems[buf]).wait()

    def fire_out(g, buf):
        jb = g * G
        for k in range(3):
            pltpu.async_copy(rows.at[buf, k],
                             out_hbm.at[k, pl.ds(pbase + jb, G)], osems[buf])

    def owait(buf):
        for k in range(3):
            pltpu.make_async_copy(rows.at[buf, k],
                                  out_hbm.at[0, pl.ds(0, G)],
                                  osems[buf]).wait()

    fire_in(0, 0)

    def pair(h, carry):
        g0 = h * 2

        @pl.when(h > 0)
        def _():
            owait(1)

        fire_in(g0 + 1, 1)
        drain_in(0)
        fire_out(g0, 0)

        @pl.when(g0 + 2 < NG)
        def _():
            owait(0)
            fire_in(g0 + 2, 0)

        drain_in(1)
        fire_out(g0 + 1, 1)
        return carry

    lax.fori_loop(0, NG // 2, pair, 0)
    owait(0)
    owait(1)


def _mlp_kernel(zg_ref, wT_ref, pfT_ref, wpT_ref, b_ref, y_ref, ps_ref,
                pss_ref):
    pf = pfT_ref[0]
    y = jnp.dot(pf, wpT_ref[...], preferred_element_type=jnp.float32)
    wT = wT_ref[0]
    for k in range(3):
        y = y + zg_ref[k] * wT[:, k:k + 1]
    y = y + b_ref[...]
    y_ref[0] = y
    ps_ref[0] = jnp.sum(y, axis=0, keepdims=True)
    pss_ref[0] = jnp.sum(y * y, axis=0, keepdims=True)


def _bn_kernel(y_ref, ac_ref, o_ref):
    yv = y_ref[0]
    z = jnp.maximum(yv * ac_ref[0:1, :] + ac_ref[1:2, :], 0.0)
    o_ref[0] = z.T


def kernel(points_coords, centers_coords, centers_features, points_features,
           W, b, gamma, beta):
    ccT = centers_coords.transpose(0, 2, 1)
    cfT = centers_features.transpose(0, 2, 1).reshape(B * M, CC)
    pfT = points_features.transpose(0, 2, 1)
    WT = W.T
    wcT = WT[:CC]
    wpT = WT[CC:]
    b_row = b.reshape(1, COUT)

    idx, wts = pl.pallas_call(
        _knn_kernel,
        grid=(B, KNB),
        in_specs=[
            pl.BlockSpec((1, 3, KBN), lambda bi, nb: (bi, 0, nb)),
            pl.BlockSpec((1, M, 3), lambda bi, nb: (bi, 0, 0)),
        ],
        out_specs=[
            pl.BlockSpec((1, 3, KBN), lambda bi, nb: (bi, 0, nb)),
            pl.BlockSpec((1, 3, KBN), lambda bi, nb: (bi, 0, nb)),
        ],
        out_shape=[
            jax.ShapeDtypeStruct((B, 3, N), jnp.int32),
            jax.ShapeDtypeStruct((B, 3, N), jnp.float32),
        ],
    )(points_coords, ccT)

    z = pl.pallas_call(
        _z_kernel,
        grid=(B,),
        in_specs=[
            pl.BlockSpec((M, CC), lambda bi: (bi, 0)),
            pl.BlockSpec((CC, COUT), lambda bi: (0, 0)),
        ],
        out_specs=pl.BlockSpec((M, COUT), lambda bi: (bi, 0)),
        out_shape=jax.ShapeDtypeStruct((B * M, COUT), jnp.float32),
    )(cfT, wcT)

    zg = _sc_gather(idx, z)

    wtsT = wts.transpose(0, 2, 1)

    yT, ps, pss = pl.pallas_call(
        _mlp_kernel,
        grid=(B, NB),
        in_specs=[
            pl.BlockSpec((3, BN, COUT), lambda bi, nb: (0, bi * NB + nb, 0)),
            pl.BlockSpec((1, BN, 3), lambda bi, nb: (bi, nb, 0)),
            pl.BlockSpec((1, BN, CP), lambda bi, nb: (bi, nb, 0)),
            pl.BlockSpec((CP, COUT), lambda bi, nb: (0, 0)),
            pl.BlockSpec((1, COUT), lambda bi, nb: (0, 0)),
        ],
        out_specs=[
            pl.BlockSpec((1, BN, COUT), lambda bi, nb: (bi, nb, 0)),
            pl.BlockSpec((1, 1, COUT), lambda bi, nb: (bi * NB + nb, 0, 0)),
            pl.BlockSpec((1, 1, COUT), lambda bi, nb: (bi * NB + nb, 0, 0)),
        ],
        out_shape=[
            jax.ShapeDtypeStruct((B, N, COUT), jnp.float32),
            jax.ShapeDtypeStruct((B * NB, 1, COUT), jnp.float32),
            jax.ShapeDtypeStruct((B * NB, 1, COUT), jnp.float32),
        ],
    )(zg, wtsT, pfT, wpT, b_row)

    cnt = jnp.float32(B * N)
    s = jnp.sum(ps[:, 0, :], axis=0)
    ss = jnp.sum(pss[:, 0, :], axis=0)
    mean = s / cnt
    var = ss / cnt - mean * mean
    a = gamma / jnp.sqrt(var + 1e-5)
    cshift = beta - mean * a
    ac = jnp.concatenate([a.reshape(1, COUT), cshift.reshape(1, COUT)], 0)

    out = pl.pallas_call(
        _bn_kernel,
        grid=(B, NB),
        in_specs=[
            pl.BlockSpec((1, BN, COUT), lambda bi, nb: (bi, nb, 0)),
            pl.BlockSpec((2, COUT), lambda bi, nb: (0, 0)),
        ],
        out_specs=pl.BlockSpec((1, COUT, BN), lambda bi, nb: (bi, 0, nb)),
        out_shape=jax.ShapeDtypeStruct((B, COUT, N), jnp.float32),
    )(yT, ac)

    return (out, points_coords)
